# baseline (speedup 1.0000x reference)
import functools

import jax
import jax.numpy as jnp
from jax import lax
from jax.experimental import pallas as pl
from jax.experimental.pallas import tpu as pltpu
from jax.experimental.pallas import tpu_sc as plsc

_VOCAB = 100000
_EMBED = 64
_BATCH = 4096
_FIELDS = 26

_NC = 2
_NS = 16
_NW = _NC * _NS

_BPW = _BATCH // _NW
_RB = 32
_NR = _BPW // _RB


@jax.jit
def _sc_gather(feat, table):
  mesh = plsc.VectorSubcoreMesh(core_axis_name="c", subcore_axis_name="s")

  @functools.partial(
      pl.kernel,
      mesh=mesh,
      compiler_params=pltpu.CompilerParams(use_tc_tiling_on_sc=False),
      out_type=jax.ShapeDtypeStruct((_BATCH, _FIELDS, _EMBED), jnp.float32),
      scratch_types=[
          pltpu.VMEM((_BPW, _FIELDS), jnp.int32),
          pltpu.VMEM((2, _RB, _FIELDS, _EMBED), jnp.float32),
          pltpu.SemaphoreType.DMA,
          pltpu.SemaphoreType.DMA,
          pltpu.SemaphoreType.DMA,
          pltpu.SemaphoreType.DMA,
      ],
  )
  def k(table_hbm, idx_hbm, out_hbm, idx_v, stage, g0, g1, o0, o1):
    sem_g = (g0, g1)
    sem_o = (o0, o1)
    wid = lax.axis_index("s") * _NC + lax.axis_index("c")
    base = wid * _BPW
    pltpu.sync_copy(idx_hbm.at[pl.ds(base, _BPW)], idx_v)

    def fire_gathers(r, p):
      for b in range(_RB):
        pltpu.async_copy(
            table_hbm.at[idx_v.at[r * _RB + b]],
            stage.at[p, b],
            sem_g[p],
        )

    def drain_gathers(p):
      for b in range(_RB):
        pltpu.make_async_copy(
            table_hbm.at[pl.ds(0, _FIELDS)], stage.at[p, b], sem_g[p]
        ).wait()

    def fire_out(r, p):
      pltpu.async_copy(
          stage.at[p], out_hbm.at[pl.ds(base + r * _RB, _RB)], sem_o[p]
      )

    def drain_out(p):
      pltpu.make_async_copy(
          stage.at[p], out_hbm.at[pl.ds(base, _RB)], sem_o[p]
      ).wait()

    fire_gathers(0, 0)
    fire_gathers(1, 1)
    for r in range(_NR):
      p = r % 2
      drain_gathers(p)
      fire_out(r, p)
      if r + 2 < _NR:
        drain_out(p)
        fire_gathers(r + 2, p)
    drain_out(0)
    drain_out(1)

  return k(table, feat)


def kernel(feat, emb_feat):
  return _sc_gather(feat, emb_feat)


# speedup vs baseline: 1.0051x; 1.0051x over previous
import functools

import jax
import jax.numpy as jnp
from jax import lax
from jax.experimental import pallas as pl
from jax.experimental.pallas import tpu as pltpu
from jax.experimental.pallas import tpu_sc as plsc

_VOCAB = 100000
_EMBED = 64
_BATCH = 4096
_FIELDS = 26

_NC = 2
_NS = 16
_NW = _NC * _NS

_BPW = _BATCH // _NW
_RB = 16
_NR = _BPW // _RB
_NBUF = 4


@jax.jit
def _sc_gather(feat, table):
  mesh = plsc.VectorSubcoreMesh(core_axis_name="c", subcore_axis_name="s")

  @functools.partial(
      pl.kernel,
      mesh=mesh,
      compiler_params=pltpu.CompilerParams(use_tc_tiling_on_sc=False),
      out_type=jax.ShapeDtypeStruct((_BATCH, _FIELDS, _EMBED), jnp.float32),
      scratch_types=[
          pltpu.VMEM((_BPW, _FIELDS), jnp.int32),
          pltpu.VMEM((_NBUF, _RB, _FIELDS, _EMBED), jnp.float32),
          [pltpu.SemaphoreType.DMA] * _NBUF,
          [pltpu.SemaphoreType.DMA] * _NBUF,
      ],
  )
  def k(table_hbm, idx_hbm, out_hbm, idx_v, stage, sem_g, sem_o):
    wid = lax.axis_index("s") * _NC + lax.axis_index("c")
    base = wid * _BPW
    pltpu.sync_copy(idx_hbm.at[pl.ds(base, _BPW)], idx_v)

    def fire_gathers(r, p):
      for b in range(_RB):
        pltpu.async_copy(
            table_hbm.at[idx_v.at[r * _RB + b]],
            stage.at[p, b],
            sem_g[p],
        )

    def drain_gathers(p):
      for b in range(_RB):
        pltpu.make_async_copy(
            table_hbm.at[pl.ds(0, _FIELDS)], stage.at[p, b], sem_g[p]
        ).wait()

    def fire_out(r, p):
      pltpu.async_copy(
          stage.at[p], out_hbm.at[pl.ds(base + r * _RB, _RB)], sem_o[p]
      )

    def drain_out(p):
      pltpu.make_async_copy(
          stage.at[p], out_hbm.at[pl.ds(base, _RB)], sem_o[p]
      ).wait()

    for p in range(_NBUF):
      fire_gathers(p, p)
    for r in range(_NR):
      p = r % _NBUF
      drain_gathers(p)
      fire_out(r, p)
      if r + _NBUF < _NR:
        drain_out(p)
        fire_gathers(r + _NBUF, p)
    for p in range(_NBUF):
      drain_out(p)

  return k(table, feat)


def kernel(feat, emb_feat):
  return _sc_gather(feat, emb_feat)
